# baseline (speedup 1.0000x reference)
import functools

import jax
import jax.numpy as jnp
from jax import lax
from jax.experimental import pallas as pl
from jax.experimental.pallas import tpu as pltpu
from jax.experimental.pallas import tpu_sc as plsc

N_CORES = 2
N_SUBCORES = 16
LANES = 16

_SC_PARAMS = pltpu.CompilerParams(needs_layout_passes=False)


def _zero_vmem(ref, n):
    zero = jnp.zeros((LANES,), jnp.float32)

    @plsc.parallel_loop(0, n, LANES, unroll=8)
    def _(i):
        ref[pl.ds(i, LANES)] = zero


def _round_bf16(v):
    b = plsc.bitcast(v, jnp.int32)
    b = (b + jnp.int32(0x7FFF) + ((b >> 16) & 1)) & jnp.int32(-65536)
    return plsc.bitcast(b, jnp.float32)


def _rsqrt16(d):
    i = plsc.bitcast(d, jnp.int32)
    i = jnp.int32(0x5F3759DF) - (i >> 1)
    y = plsc.bitcast(i, jnp.float32)
    for _ in range(3):
        y = y * (1.5 - 0.5 * d * y * y)
    return y


def _reduce_stripe(shared, tmps, dst, sem, st, width, npad):
    descs = [
        pltpu.async_copy(
            shared.at[pl.ds(j * npad + st, width)],
            tmps.at[pl.ds(j * width, width)], sem)
        for j in range(N_SUBCORES)
    ]
    for dsc in descs:
        dsc.wait()

    @plsc.parallel_loop(0, width, LANES, unroll=4)
    def _(k):
        v = tmps[pl.ds(k, LANES)]
        for j in range(1, N_SUBCORES):
            v = v + tmps[pl.ds(j * width + k, LANES)]
        dst[pl.ds(k, LANES)] = v


def _publish_reduce_emit(acc, shared, out_hbm, tmps, accr, sem, sid, npad, out_base):
    stripe = npad // N_SUBCORES
    pltpu.sync_copy(acc, shared.at[pl.ds(sid * npad, npad)])
    plsc.subcore_barrier()
    st = sid * stripe
    _reduce_stripe(shared, tmps, accr, sem, st, stripe, npad)
    pltpu.sync_copy(accr, out_hbm.at[pl.ds(out_base + st, stripe)])
    plsc.subcore_barrier()


def _make_m1_kernel(e, npad):
    ch = e // N_SUBCORES
    stripe = npad // N_SUBCORES
    mesh = plsc.VectorSubcoreMesh(core_axis_name="c", subcore_axis_name="s")

    @functools.partial(
        pl.kernel,
        out_type=jax.ShapeDtypeStruct((9 * npad,), jnp.float32),
        mesh=mesh,
        compiler_params=_SC_PARAMS,
        scratch_types=[
            pltpu.VMEM((ch,), jnp.int32),
            pltpu.VMEM((ch,), jnp.int32),
            pltpu.VMEM((npad,), jnp.float32),
            pltpu.VMEM((npad,), jnp.float32),
            pltpu.VMEM((npad,), jnp.float32),
            pltpu.VMEM((npad,), jnp.float32),
            pltpu.VMEM((npad,), jnp.float32),
            pltpu.VMEM((npad,), jnp.float32),
            pltpu.VMEM((N_SUBCORES * stripe,), jnp.float32),
            pltpu.VMEM((stripe,), jnp.float32),
            pltpu.VMEM((stripe,), jnp.float32),
            pltpu.VMEM((stripe,), jnp.int32),
            pltpu.VMEM((stripe,), jnp.float32),
            pltpu.VMEM((stripe,), jnp.float32),
            pltpu.VMEM((stripe,), jnp.float32),
            pltpu.VMEM((LANES,), jnp.float32),
            pltpu.VMEM_SHARED((N_SUBCORES * npad,), jnp.float32),
            pltpu.VMEM_SHARED((3 * npad,), jnp.float32),
            pltpu.SemaphoreType.DMA,
        ],
    )
    def m1(edges_hbm, w_hbm, sig_hbm, et_hbm, out_hbm,
           src_v, dst_v, g0, g1, g2, a0, a1, a2, tmps, accr,
           wst, sigst, gst, gst1, gst2, et_v, shared, sharedg, sem):
        cid = lax.axis_index("c")
        sid = lax.axis_index("s")
        st = sid * stripe
        pltpu.sync_copy(edges_hbm.at[pl.ds(sid * ch, ch)], src_v)
        pltpu.sync_copy(edges_hbm.at[pl.ds(e + sid * ch, ch)], dst_v)
        pltpu.sync_copy(et_hbm.at[pl.ds(0, LANES)], et_v)
        pltpu.sync_copy(w_hbm.at[pl.ds(st, stripe)], wst)
        pltpu.sync_copy(sig_hbm.at[pl.ds(st, stripe)], sigst)

        _zero_vmem(a0, npad)
        ones = jnp.full((LANES,), 1.0, jnp.float32)

        @plsc.parallel_loop(0, ch, LANES, unroll=8)
        def _(i):
            ix = src_v[pl.ds(i, LANES)]
            plsc.addupdate_scatter(a0, [ix], ones)

        pltpu.sync_copy(a0, shared.at[pl.ds(sid * npad, npad)])
        plsc.subcore_barrier()
        _reduce_stripe(shared, tmps, accr, sem, st, stripe, npad)

        zero16 = jnp.zeros((LANES,), jnp.int32)
        et00 = plsc.load_gather(et_v, [zero16 + 1])
        et01 = plsc.load_gather(et_v, [zero16 + 2])
        et10 = plsc.load_gather(et_v, [zero16 + 3])
        et11 = plsc.load_gather(et_v, [zero16 + 4])

        et00 = _round_bf16(et00)
        et01 = _round_bf16(et01)
        et10 = _round_bf16(et10)
        et11 = _round_bf16(et11)

        @plsc.parallel_loop(0, stripe, LANES, unroll=2)
        def _(k):
            r = _rsqrt16(accr[pl.ds(k, LANES)] + 1.0)
            sigf = sigst[pl.ds(k, LANES)].astype(jnp.float32)
            wr = _round_bf16(wst[pl.ds(k, LANES)])
            accr[pl.ds(k, LANES)] = r
            gst[pl.ds(k, LANES)] = wr * r
            gst1[pl.ds(k, LANES)] = (et00 + sigf * (et10 - et00)) * r
            gst2[pl.ds(k, LANES)] = (et01 + sigf * (et11 - et01)) * r

        @pl.when(cid == 0)
        def _():
            pltpu.sync_copy(gst, sharedg.at[pl.ds(st, stripe)])
            pltpu.sync_copy(gst, out_hbm.at[pl.ds(5 * npad + st, stripe)])
            pltpu.sync_copy(gst1, sharedg.at[pl.ds(npad + st, stripe)])
            pltpu.sync_copy(gst1, out_hbm.at[pl.ds(6 * npad + st, stripe)])
            pltpu.sync_copy(gst2, sharedg.at[pl.ds(2 * npad + st, stripe)])

        @pl.when(cid == 1)
        def _():
            pltpu.sync_copy(accr, sharedg.at[pl.ds(st, stripe)])
            pltpu.sync_copy(accr, out_hbm.at[pl.ds(8 * npad + st, stripe)])
            pltpu.sync_copy(gst2, out_hbm.at[pl.ds(7 * npad + st, stripe)])

        plsc.subcore_barrier()

        pltpu.sync_copy(sharedg.at[pl.ds(0, npad)], g0)
        _zero_vmem(a0, npad)
        _zero_vmem(a1, npad)

        @pl.when(cid == 0)
        def _():
            pltpu.sync_copy(sharedg.at[pl.ds(npad, npad)], g1)
            pltpu.sync_copy(sharedg.at[pl.ds(2 * npad, npad)], g2)
            _zero_vmem(a2, npad)

        plsc.subcore_barrier()

        @pl.when(cid == 0)
        def _():
            @plsc.parallel_loop(0, ch, LANES, unroll=4)
            def _(i):
                ix_s = src_v[pl.ds(i, LANES)]
                ix_d = dst_v[pl.ds(i, LANES)]
                v0 = plsc.load_gather(g0, [ix_s])
                v1 = plsc.load_gather(g1, [ix_s])
                v2 = plsc.load_gather(g2, [ix_s])
                plsc.addupdate_scatter(a0, [ix_d], v0)
                plsc.addupdate_scatter(a1, [ix_d], v1)
                plsc.addupdate_scatter(a2, [ix_d], v2)

            _publish_reduce_emit(a0, shared, out_hbm, tmps, accr, sem, sid, npad, 0)
            _publish_reduce_emit(a1, shared, out_hbm, tmps, accr, sem, sid, npad, npad)
            _publish_reduce_emit(a2, shared, out_hbm, tmps, accr, sem, sid, npad, 2 * npad)

        @pl.when(cid == 1)
        def _():
            @plsc.parallel_loop(0, ch, LANES, unroll=4)
            def _(i):
                ix_s = src_v[pl.ds(i, LANES)]
                ix_d = dst_v[pl.ds(i, LANES)]
                v3 = plsc.load_gather(g0, [ix_s])
                plsc.addupdate_scatter(a0, [ix_d], v3)
                plsc.addupdate_scatter(a1, [ix_d], ones)

            _publish_reduce_emit(a0, shared, out_hbm, tmps, accr, sem, sid, npad, 3 * npad)
            _publish_reduce_emit(a1, shared, out_hbm, tmps, accr, sem, sid, npad, 4 * npad)

    return m1


def _make_m2_kernel(e, npad, d):
    n_ch = d // (N_CORES * N_SUBCORES)
    ec = 16000
    n_chunks = e // ec
    mesh = plsc.VectorSubcoreMesh(core_axis_name="c", subcore_axis_name="s")

    scratch = [
        pltpu.VMEM((ec,), jnp.int32),
        pltpu.VMEM((ec,), jnp.int32),
    ]
    for _ in range(2 * n_ch):
        scratch.append(pltpu.VMEM((npad,), jnp.float32))

    @functools.partial(
        pl.kernel,
        out_type=jax.ShapeDtypeStruct((d * npad,), jnp.float32),
        mesh=mesh,
        compiler_params=_SC_PARAMS,
        scratch_types=scratch,
    )
    def m2(edges_hbm, h_hbm, out_hbm, src_v, dst_v, *hs_as):
        hs = hs_as[:n_ch]
        accs = hs_as[n_ch:]
        cid = lax.axis_index("c")
        sid = lax.axis_index("s")
        w = sid * N_CORES + cid
        cbase = w * n_ch
        for j in range(n_ch):
            pltpu.sync_copy(h_hbm.at[pl.ds((cbase + j) * npad, npad)], hs[j])
            _zero_vmem(accs[j], npad)

        def chunk(c, _):
            pltpu.sync_copy(edges_hbm.at[pl.ds(c * ec, ec)], src_v)
            pltpu.sync_copy(edges_hbm.at[pl.ds(e + c * ec, ec)], dst_v)

            @plsc.parallel_loop(0, ec, LANES, unroll=4)
            def _(i):
                ix_s = src_v[pl.ds(i, LANES)]
                ix_d = dst_v[pl.ds(i, LANES)]
                for j in range(n_ch):
                    v = plsc.load_gather(hs[j], [ix_s])
                    plsc.addupdate_scatter(accs[j], [ix_d], v)

            return 0

        lax.fori_loop(0, n_chunks, chunk, 0)
        for j in range(n_ch):
            pltpu.sync_copy(accs[j], out_hbm.at[pl.ds((cbase + j) * npad, npad)])

    return m2


def _dense0_kernel(agg_ref, din_ref, g_ref, wlin_ref, wc0_ref,
                   blin_ref, bc0_ref, h1_ref, rin_ref):
    rin = lax.rsqrt(din_ref[...] + 1.0)
    a = (agg_ref[...] + g_ref[...]) * rin
    a3 = a[0:3, :]
    s = a[3:4, :]
    dn = (((1,), (0,)), ((), ()))
    wlb = wlin_ref[...].astype(jnp.bfloat16).astype(jnp.float32)
    t1 = lax.dot_general(wlb, a3, dn, preferred_element_type=jnp.float32,
                         precision=lax.Precision.HIGHEST)
    t1 = t1 + blin_ref[...] * s
    emb = lax.dot_general(wc0_ref[...], t1, dn,
                          preferred_element_type=jnp.float32,
                          precision=lax.Precision.DEFAULT)
    emb = emb + bc0_ref[...]
    h = jnp.where(emb >= 0.0, emb, 0.01 * emb)
    h1_ref[...] = h * g_ref[3:4, :]
    rin_ref[...] = rin


def _dense1_kernel(agg_ref, h1_ref, rin_ref, wc1_ref, wp_ref,
                   bc1_ref, bp_ref, out_ref):
    a = (agg_ref[...] + h1_ref[...]) * rin_ref[...]
    dn = (((1,), (0,)), ((), ()))
    emb = lax.dot_general(wc1_ref[...], a, dn,
                          preferred_element_type=jnp.float32,
                          precision=lax.Precision.DEFAULT)
    emb = emb + bc1_ref[...]
    out_ref[...] = lax.dot_general(
        wp_ref[...], emb, dn, preferred_element_type=jnp.float32,
        precision=lax.Precision.DEFAULT) + bp_ref[0, 0]


def kernel(significance, weight, edge_index, embed_table, W_lin, b_lin,
           W_conv0, b_conv0, W_conv1, b_conv1, W_pred, b_pred):
    n = significance.shape[0]
    e = edge_index.shape[1]
    d = W_lin.shape[0]
    npad = ((n + 2047) // 2048) * 2048

    edges = edge_index.astype(jnp.int32).reshape(-1)
    wpad = jnp.pad(weight.astype(jnp.float32), (0, npad - n))
    sigpad = jnp.pad(significance.astype(jnp.int32), (0, npad - n))
    etflat = jnp.pad(embed_table.astype(jnp.float32).reshape(-1), (1, LANES - 5))

    m1_out = _make_m1_kernel(e, npad)(edges, wpad, sigpad, etflat)
    agg = m1_out[0:4 * npad].reshape(4, npad)
    din = m1_out[4 * npad:5 * npad].reshape(1, npad)
    g = m1_out[5 * npad:9 * npad].reshape(4, npad)

    bw = 512
    grid = (npad // bw,)
    h1, rin = pl.pallas_call(
        _dense0_kernel,
        grid=grid,
        out_shape=(
            jax.ShapeDtypeStruct((d, npad), jnp.float32),
            jax.ShapeDtypeStruct((1, npad), jnp.float32),
        ),
        in_specs=[
            pl.BlockSpec((4, bw), lambda i: (0, i)),
            pl.BlockSpec((1, bw), lambda i: (0, i)),
            pl.BlockSpec((4, bw), lambda i: (0, i)),
            pl.BlockSpec((d, 3), lambda i: (0, 0)),
            pl.BlockSpec((d, d), lambda i: (0, 0)),
            pl.BlockSpec((d, 1), lambda i: (0, 0)),
            pl.BlockSpec((d, 1), lambda i: (0, 0)),
        ],
        out_specs=(
            pl.BlockSpec((d, bw), lambda i: (0, i)),
            pl.BlockSpec((1, bw), lambda i: (0, i)),
        ),
    )(
        agg, din, g,
        W_lin.astype(jnp.float32), W_conv0.astype(jnp.float32),
        b_lin.astype(jnp.float32).reshape(d, 1),
        b_conv0.astype(jnp.float32).reshape(d, 1),
    )

    agg1 = _make_m2_kernel(e, npad, d)(edges, h1.reshape(-1)).reshape(d, npad)

    out = pl.pallas_call(
        _dense1_kernel,
        grid=grid,
        out_shape=jax.ShapeDtypeStruct((1, npad), jnp.float32),
        in_specs=[
            pl.BlockSpec((d, bw), lambda i: (0, i)),
            pl.BlockSpec((d, bw), lambda i: (0, i)),
            pl.BlockSpec((1, bw), lambda i: (0, i)),
            pl.BlockSpec((d, d), lambda i: (0, 0)),
            pl.BlockSpec((1, d), lambda i: (0, 0)),
            pl.BlockSpec((d, 1), lambda i: (0, 0)),
            pl.BlockSpec(memory_space=pltpu.SMEM),
        ],
        out_specs=pl.BlockSpec((1, bw), lambda i: (0, i)),
    )(
        agg1, h1, rin,
        W_conv1.astype(jnp.float32), W_pred.astype(jnp.float32),
        b_conv1.astype(jnp.float32).reshape(d, 1),
        b_pred.astype(jnp.float32).reshape(1, 1),
    )

    return jnp.reshape(out[0, :n], (n, 1))
